# baseline (speedup 1.0000x reference)
import jax
import jax.numpy as jnp
from jax import lax
from jax.experimental import pallas as pl
from jax.experimental.pallas import tpu as pltpu
from jax.experimental.pallas import tpu_sc as plsc

N = 10000
NP = 10240
E = 320000
EP = 327680
PAD_E = EP - E
RB = 640
NBLK = NP // RB
CHW = 128
NCH_T = 160
NCH_H = 80
F_IN = 128
H1 = 256
H2 = 196
DIM_OUT = 10
G = 64
FEPS = 1e-5


def _hist_body(dst_hbm, zeros_hbm, ones_hbm, out_hbm, idx_v, ones_v, deg_sp):
    c = lax.axis_index("c")
    s = lax.axis_index("s")
    w = c * 16 + s
    pltpu.sync_copy(ones_hbm, ones_v)
    pltpu.sync_copy(zeros_hbm, deg_sp.at[pl.ds(s * RB, RB)])
    pltpu.sync_copy(dst_hbm.at[pl.ds(w * NCH_H, NCH_H)], idx_v)
    plsc.subcore_barrier()

    @pl.loop(0, NCH_H)
    def _(j):
        pltpu.sync_copy(ones_v, deg_sp.at[idx_v.at[j]], add=True)

    plsc.subcore_barrier()
    pltpu.sync_copy(deg_sp.at[pl.ds(s * RB, RB)],
                    out_hbm.at[pl.ds(c * NP + s * RB, RB)])


def _make_hist():
    return pl.kernel(
        _hist_body,
        out_type=jax.ShapeDtypeStruct((2 * NP, CHW), jnp.float32),
        mesh=plsc.VectorSubcoreMesh(core_axis_name="c", subcore_axis_name="s"),
        scratch_types=[
            pltpu.VMEM((NCH_H, CHW), jnp.int32),
            pltpu.VMEM((CHW, CHW), jnp.float32),
            pltpu.VMEM_SHARED((NP, CHW), jnp.float32),
        ],
    )


GSZ = 16
NGRP = NCH_T // GSZ


def _mp_body(y_hbm, src_hbm, dst_hbm, out_hbm, srcv, dstv, rows0, rows1,
             acc_sp, gsem0, gsem1, ssem0, ssem1):
    c = lax.axis_index("c")
    s = lax.axis_index("s")
    base = c * NP + s * RB
    pltpu.sync_copy(y_hbm.at[pl.ds(base, RB)], acc_sp.at[pl.ds(s * RB, RB)])
    w = c * 16 + s
    plsc.subcore_barrier()

    @pl.loop(0, NGRP)
    def _(g):
        pltpu.sync_copy(src_hbm.at[pl.ds(w * NCH_T + g * GSZ, GSZ)], srcv)
        pltpu.sync_copy(dst_hbm.at[pl.ds(s * NCH_T + g * GSZ, GSZ)], dstv)
        pltpu.async_copy(y_hbm.at[srcv.at[0]], rows0, gsem0)
        for j in range(GSZ):
            even = j % 2 == 0
            rcur, gcur, scur = ((rows0, gsem0, ssem0) if even
                                else (rows1, gsem1, ssem1))
            rnxt, gnxt, snxt = ((rows1, gsem1, ssem1) if even
                                else (rows0, gsem0, ssem0))
            pltpu.make_async_copy(y_hbm.at[srcv.at[j]], rcur, gcur).wait()
            if j >= 1:
                pltpu.make_async_copy(
                    rnxt, acc_sp.at[dstv.at[j - 1]], snxt).wait()
            if j + 1 < GSZ:
                pltpu.async_copy(y_hbm.at[srcv.at[j + 1]], rnxt, gnxt)
            pltpu.async_copy(rcur, acc_sp.at[dstv.at[j]], scur, add=True)
        rl, sl = (rows0, ssem0) if (GSZ - 1) % 2 == 0 else (rows1, ssem1)
        pltpu.make_async_copy(rl, acc_sp.at[dstv.at[GSZ - 1]], sl).wait()

    plsc.subcore_barrier()
    pltpu.sync_copy(acc_sp.at[pl.ds(s * RB, RB)], out_hbm.at[pl.ds(base, RB)])


def _make_mp():
    return pl.kernel(
        _mp_body,
        out_type=jax.ShapeDtypeStruct((2 * NP, CHW), jnp.float32),
        mesh=plsc.VectorSubcoreMesh(core_axis_name="c", subcore_axis_name="s"),
        scratch_types=[
            pltpu.VMEM((GSZ, CHW), jnp.int32),
            pltpu.VMEM((GSZ, CHW), jnp.int32),
            pltpu.VMEM((CHW, CHW), jnp.float32),
            pltpu.VMEM((CHW, CHW), jnp.float32),
            pltpu.VMEM_SHARED((NP, CHW), jnp.float32),
            pltpu.SemaphoreType.DMA,
            pltpu.SemaphoreType.DMA,
            pltpu.SemaphoreType.DMA,
            pltpu.SemaphoreType.DMA,
        ],
    )


def _sc_kernels():
    return _make_hist(), _make_mp()


def _dinv_of(degp):
    return lax.rsqrt(degp[0, :, 0] + degp[1, :, 0] + 1.0)


def _mm1_body(x_ref, w_ref, degp_ref, out_ref):
    dinv = _dinv_of(degp_ref[...])
    h = jnp.dot(x_ref[...], w_ref[...], preferred_element_type=jnp.float32)
    y = h * dinv[:, None]
    out_ref[0] = y[:, :CHW]
    out_ref[1] = y[:, CHW:]


def _mm1(xp, W1, degp):
    return pl.pallas_call(
        _mm1_body,
        grid=(NBLK,),
        in_specs=[
            pl.BlockSpec((RB, F_IN), lambda i: (i, 0)),
            pl.BlockSpec((F_IN, H1), lambda i: (0, 0)),
            pl.BlockSpec((2, RB, CHW), lambda i: (0, i, 0)),
        ],
        out_specs=pl.BlockSpec((2, RB, CHW), lambda i: (0, i, 0)),
        out_shape=jax.ShapeDtypeStruct((2, NP, CHW), jnp.float32),
    )(xp, W1, degp)


def _zstats_body(acc_ref, degp_ref, b_ref, z_ref, st_ref, ssum, ssq, kref):
    i = pl.program_id(0)
    dinv = _dinv_of(degp_ref[...])
    z = (jnp.concatenate([acc_ref[0], acc_ref[1]], axis=1) * dinv[:, None]
         + b_ref[...])
    z_ref[...] = z

    @pl.when(i == 0)
    def _():
        kref[...] = jnp.sum(z, axis=0, keepdims=True) / float(RB)

    rid = lax.broadcasted_iota(jnp.int32, (RB, 1), 0) + i * RB
    d = jnp.where(rid < N, z - kref[...], 0.0)
    ps = jnp.sum(d, axis=0, keepdims=True)
    pq = jnp.sum(d * d, axis=0, keepdims=True)

    @pl.when(i == 0)
    def _():
        ssum[...] = ps
        ssq[...] = pq

    @pl.when(i > 0)
    def _():
        ssum[...] += ps
        ssq[...] += pq

    @pl.when(i == NBLK - 1)
    def _():
        st_ref[...] = jnp.concatenate([ssum[...], ssq[...], kref[...]],
                                      axis=0)


def _zstats(acc, degp, b):
    return pl.pallas_call(
        _zstats_body,
        grid=(NBLK,),
        in_specs=[
            pl.BlockSpec((2, RB, CHW), lambda i: (0, i, 0)),
            pl.BlockSpec((2, RB, CHW), lambda i: (0, i, 0)),
            pl.BlockSpec((1, H1), lambda i: (0, 0)),
        ],
        out_specs=[
            pl.BlockSpec((RB, H1), lambda i: (i, 0)),
            pl.BlockSpec((3, H1), lambda i: (0, 0)),
        ],
        out_shape=[
            jax.ShapeDtypeStruct((NP, H1), jnp.float32),
            jax.ShapeDtypeStruct((3, H1), jnp.float32),
        ],
        scratch_shapes=[
            pltpu.VMEM((1, H1), jnp.float32),
            pltpu.VMEM((1, H1), jnp.float32),
            pltpu.VMEM((1, H1), jnp.float32),
        ],
    )(acc, degp, b)


def _bn_relu(z, st, g, bt):
    m = st[0:1] / float(N)
    mu = m + st[2:3]
    var = st[1:2] / float(N) - m * m
    inv = lax.rsqrt(var + FEPS)
    return jnp.maximum((z - mu) * inv * g + bt, 0.0)


def _bnmm_body(z_ref, st_ref, g_ref, bt_ref, w_ref, degp_ref, out_ref):
    h = _bn_relu(z_ref[...], st_ref[...], g_ref[...], bt_ref[...])
    dinv = _dinv_of(degp_ref[...])
    y = jnp.dot(h, w_ref[...], preferred_element_type=jnp.float32)
    y = y * dinv[:, None]
    out_ref[0] = y[:, :CHW]
    out_ref[1] = y[:, CHW:]


def _bnmm(z, st, g, bt, W2, degp):
    return pl.pallas_call(
        _bnmm_body,
        grid=(NBLK,),
        in_specs=[
            pl.BlockSpec((RB, H1), lambda i: (i, 0)),
            pl.BlockSpec((3, H1), lambda i: (0, 0)),
            pl.BlockSpec((1, H1), lambda i: (0, 0)),
            pl.BlockSpec((1, H1), lambda i: (0, 0)),
            pl.BlockSpec((H1, H1), lambda i: (0, 0)),
            pl.BlockSpec((2, RB, CHW), lambda i: (0, i, 0)),
        ],
        out_specs=pl.BlockSpec((2, RB, CHW), lambda i: (0, i, 0)),
        out_shape=jax.ShapeDtypeStruct((2, NP, CHW), jnp.float32),
    )(z, st, g, bt, W2, degp)


def _pool_body(z_ref, st_ref, g_ref, bt_ref, batch_ref, ps_ref, pc_ref,
               asum, acnt):
    i = pl.program_id(0)
    h = _bn_relu(z_ref[...], st_ref[...], g_ref[...], bt_ref[...])
    b = batch_ref[0, 0, :]
    oh = (lax.broadcasted_iota(jnp.int32, (G, RB), 0) == b[None, :]).astype(
        jnp.float32)
    ps = jnp.dot(oh, h, preferred_element_type=jnp.float32,
                 precision=lax.Precision.HIGHEST)
    pc = jnp.dot(oh, jnp.ones((RB, CHW), jnp.float32),
                 preferred_element_type=jnp.float32)

    @pl.when(i == 0)
    def _():
        asum[...] = ps
        acnt[...] = pc

    @pl.when(i > 0)
    def _():
        asum[...] += ps
        acnt[...] += pc

    @pl.when(i == NBLK - 1)
    def _():
        ps_ref[...] = asum[...]
        pc_ref[...] = acnt[...]


def _pool(z, st, g, bt, batch3):
    return pl.pallas_call(
        _pool_body,
        grid=(NBLK,),
        in_specs=[
            pl.BlockSpec((RB, H1), lambda i: (i, 0)),
            pl.BlockSpec((3, H1), lambda i: (0, 0)),
            pl.BlockSpec((1, H1), lambda i: (0, 0)),
            pl.BlockSpec((1, H1), lambda i: (0, 0)),
            pl.BlockSpec((1, 1, RB), lambda i: (i, 0, 0)),
        ],
        out_specs=[
            pl.BlockSpec((G, H1), lambda i: (0, 0)),
            pl.BlockSpec((G, CHW), lambda i: (0, 0)),
        ],
        out_shape=[
            jax.ShapeDtypeStruct((G, H1), jnp.float32),
            jax.ShapeDtypeStruct((G, CHW), jnp.float32),
        ],
        scratch_shapes=[
            pltpu.VMEM((G, H1), jnp.float32),
            pltpu.VMEM((G, CHW), jnp.float32),
        ],
    )(z, st, g, bt, batch3)


def _head_body(ps_ref, pc_ref, wf2_ref, bf2_ref, g3_ref, b3_ref, wf3_ref,
               bf3_ref, out_ref, hg_ref):
    cnt = pc_ref[...][:, 0:1]
    hg = ps_ref[...] / jnp.maximum(cnt, 1.0)
    hg_ref[...] = hg
    a = jnp.dot(hg, wf2_ref[...], preferred_element_type=jnp.float32)
    a = a + bf2_ref[...]
    mu = jnp.mean(a, axis=0, keepdims=True)
    ac = a - mu
    var = jnp.mean(ac * ac, axis=0, keepdims=True)
    h2 = jnp.maximum((a - mu) * lax.rsqrt(var + FEPS) * g3_ref[...]
                     + b3_ref[...], 0.0)
    out_ref[...] = (jnp.dot(h2, wf3_ref[...], preferred_element_type=jnp.float32)
                    + bf3_ref[...])


def _head(ps, pc, Wf2, bf2, g3, b3, Wf3, bf3):
    return pl.pallas_call(
        _head_body,
        out_shape=[
            jax.ShapeDtypeStruct((G, DIM_OUT), jnp.float32),
            jax.ShapeDtypeStruct((G, H1), jnp.float32),
        ],
    )(ps, pc, Wf2, bf2, g3, b3, Wf3, bf3)


def kernel(x, edge_index, batch, W1, b1, gamma1, beta1, W2, b2, gamma2, beta2,
           Wf2, bf2, gamma3, beta3, Wf3, bf3):
    src = edge_index[0]
    dst = edge_index[1]
    srcp = jnp.concatenate([src, jnp.zeros((PAD_E,), jnp.int32)])
    dstp = jnp.concatenate([dst, jnp.full((PAD_E,), N, jnp.int32)])
    src_r = srcp.reshape(16, NCH_T, CHW)
    src2 = jnp.concatenate([src_r, src_r + NP], axis=0).reshape(
        32 * NCH_T, CHW)
    dst2 = dstp.reshape(16 * NCH_T, CHW)
    xp = jnp.pad(x, ((0, NP - N), (0, 0)))
    batch3 = jnp.pad(batch, (0, NP - N), constant_values=G).reshape(
        NBLK, 1, RB)
    zeros_h = jnp.zeros((RB, CHW), jnp.float32)
    ones_h = jnp.ones((CHW, CHW), jnp.float32)

    hist_k, mp_k = _sc_kernels()
    degp = hist_k(dst2, zeros_h, ones_h).reshape(2, NP, CHW)
    y1 = _mm1(xp, W1, degp)
    acc1 = mp_k(y1.reshape(2 * NP, CHW), src2, dst2).reshape(2, NP, CHW)
    z1, st1 = _zstats(acc1, degp, b1.reshape(1, H1))
    y2 = _bnmm(z1, st1, gamma1.reshape(1, H1), beta1.reshape(1, H1), W2, degp)
    acc2 = mp_k(y2.reshape(2 * NP, CHW), src2, dst2).reshape(2, NP, CHW)
    z2, st2 = _zstats(acc2, degp, b2.reshape(1, H1))
    ps, pc = _pool(z2, st2, gamma2.reshape(1, H1), beta2.reshape(1, H1),
                   batch3)
    out, hg = _head(ps, pc, Wf2, bf2.reshape(1, H2), gamma3.reshape(1, H2),
                    beta3.reshape(1, H2), Wf3, bf3.reshape(1, DIM_OUT))
    return (out, hg)


# speedup vs baseline: 1.0130x; 1.0130x over previous
import jax
import jax.numpy as jnp
from jax import lax
from jax.experimental import pallas as pl
from jax.experimental.pallas import tpu as pltpu
from jax.experimental.pallas import tpu_sc as plsc

N = 10000
NP = 10240
E = 320000
EP = 327680
PAD_E = EP - E
RB = 640
NBLK = NP // RB
CHW = 128
NCH_T = 160
NCH_H = 80
F_IN = 128
H1 = 256
H2 = 196
DIM_OUT = 10
G = 64
FEPS = 1e-5


def _hist_body(dst_hbm, zeros_hbm, ones_hbm, out_hbm, idx_v, ones_v, deg_sp):
    c = lax.axis_index("c")
    s = lax.axis_index("s")
    w = c * 16 + s
    pltpu.sync_copy(ones_hbm, ones_v)
    pltpu.sync_copy(zeros_hbm, deg_sp.at[pl.ds(s * RB, RB)])
    pltpu.sync_copy(dst_hbm.at[pl.ds(w * NCH_H, NCH_H)], idx_v)
    plsc.subcore_barrier()

    @pl.loop(0, NCH_H)
    def _(j):
        pltpu.sync_copy(ones_v, deg_sp.at[idx_v.at[j]], add=True)

    plsc.subcore_barrier()
    pltpu.sync_copy(deg_sp.at[pl.ds(s * RB, RB)],
                    out_hbm.at[pl.ds(c * NP + s * RB, RB)])


def _make_hist():
    return pl.kernel(
        _hist_body,
        out_type=jax.ShapeDtypeStruct((2 * NP, CHW), jnp.float32),
        mesh=plsc.VectorSubcoreMesh(core_axis_name="c", subcore_axis_name="s"),
        scratch_types=[
            pltpu.VMEM((NCH_H, CHW), jnp.int32),
            pltpu.VMEM((CHW, CHW), jnp.float32),
            pltpu.VMEM_SHARED((NP, CHW), jnp.float32),
        ],
    )


GSZ = 16
NGRP = NCH_T // GSZ


def _mp_body(y_hbm, src_hbm, dst_hbm, out_hbm, srcv0, dstv0, srcv1, dstv1,
             rows0, rows1, acc_sp, gsem0, gsem1, ssem0, ssem1, isem):
    c = lax.axis_index("c")
    s = lax.axis_index("s")
    base = c * NP + s * RB
    pltpu.sync_copy(y_hbm.at[pl.ds(base, RB)], acc_sp.at[pl.ds(s * RB, RB)])
    w = c * 16 + s
    plsc.subcore_barrier()

    def run_group(srcv, dstv):
        pltpu.async_copy(y_hbm.at[srcv.at[0]], rows0, gsem0)
        for j in range(GSZ):
            even = j % 2 == 0
            rcur, gcur, scur = ((rows0, gsem0, ssem0) if even
                                else (rows1, gsem1, ssem1))
            rnxt, gnxt, snxt = ((rows1, gsem1, ssem1) if even
                                else (rows0, gsem0, ssem0))
            pltpu.make_async_copy(y_hbm.at[srcv.at[j]], rcur, gcur).wait()
            if j >= 1:
                pltpu.make_async_copy(
                    rnxt, acc_sp.at[dstv.at[j - 1]], snxt).wait()
            if j + 1 < GSZ:
                pltpu.async_copy(y_hbm.at[srcv.at[j + 1]], rnxt, gnxt)
            pltpu.async_copy(rcur, acc_sp.at[dstv.at[j]], scur, add=True)
        rl, sl = (rows0, ssem0) if (GSZ - 1) % 2 == 0 else (rows1, ssem1)
        pltpu.make_async_copy(rl, acc_sp.at[dstv.at[GSZ - 1]], sl).wait()

    pltpu.sync_copy(src_hbm.at[pl.ds(w * NCH_T, GSZ)], srcv0)
    pltpu.sync_copy(dst_hbm.at[pl.ds(s * NCH_T, GSZ)], dstv0)

    @pl.loop(0, NGRP, step=2)
    def _(g):
        pltpu.async_copy(src_hbm.at[pl.ds(w * NCH_T + (g + 1) * GSZ, GSZ)],
                         srcv1, isem)
        pltpu.async_copy(dst_hbm.at[pl.ds(s * NCH_T + (g + 1) * GSZ, GSZ)],
                         dstv1, isem)
        run_group(srcv0, dstv0)
        pltpu.make_async_copy(src_hbm.at[pl.ds(0, GSZ)], srcv1, isem).wait()
        pltpu.make_async_copy(dst_hbm.at[pl.ds(0, GSZ)], dstv1, isem).wait()
        g2 = lax.rem(g + 2, NGRP)
        pltpu.async_copy(src_hbm.at[pl.ds(w * NCH_T + g2 * GSZ, GSZ)],
                         srcv0, isem)
        pltpu.async_copy(dst_hbm.at[pl.ds(s * NCH_T + g2 * GSZ, GSZ)],
                         dstv0, isem)
        run_group(srcv1, dstv1)
        pltpu.make_async_copy(src_hbm.at[pl.ds(0, GSZ)], srcv0, isem).wait()
        pltpu.make_async_copy(dst_hbm.at[pl.ds(0, GSZ)], dstv0, isem).wait()

    plsc.subcore_barrier()
    pltpu.sync_copy(acc_sp.at[pl.ds(s * RB, RB)], out_hbm.at[pl.ds(base, RB)])


def _make_mp():
    return pl.kernel(
        _mp_body,
        out_type=jax.ShapeDtypeStruct((2 * NP, CHW), jnp.float32),
        mesh=plsc.VectorSubcoreMesh(core_axis_name="c", subcore_axis_name="s"),
        scratch_types=[
            pltpu.VMEM((GSZ, CHW), jnp.int32),
            pltpu.VMEM((GSZ, CHW), jnp.int32),
            pltpu.VMEM((GSZ, CHW), jnp.int32),
            pltpu.VMEM((GSZ, CHW), jnp.int32),
            pltpu.VMEM((CHW, CHW), jnp.float32),
            pltpu.VMEM((CHW, CHW), jnp.float32),
            pltpu.VMEM_SHARED((NP, CHW), jnp.float32),
            pltpu.SemaphoreType.DMA,
            pltpu.SemaphoreType.DMA,
            pltpu.SemaphoreType.DMA,
            pltpu.SemaphoreType.DMA,
            pltpu.SemaphoreType.DMA,
        ],
    )


def _sc_kernels():
    return _make_hist(), _make_mp()


def _dinv_of(degp):
    return lax.rsqrt(degp[0, :, 0] + degp[1, :, 0] + 1.0)


def _mm1_body(x_ref, w_ref, degp_ref, out_ref):
    dinv = _dinv_of(degp_ref[...])
    h = jnp.dot(x_ref[...], w_ref[...], preferred_element_type=jnp.float32)
    y = h * dinv[:, None]
    out_ref[0] = y[:, :CHW]
    out_ref[1] = y[:, CHW:]


def _mm1(xp, W1, degp):
    return pl.pallas_call(
        _mm1_body,
        grid=(NBLK,),
        in_specs=[
            pl.BlockSpec((RB, F_IN), lambda i: (i, 0)),
            pl.BlockSpec((F_IN, H1), lambda i: (0, 0)),
            pl.BlockSpec((2, RB, CHW), lambda i: (0, i, 0)),
        ],
        out_specs=pl.BlockSpec((2, RB, CHW), lambda i: (0, i, 0)),
        out_shape=jax.ShapeDtypeStruct((2, NP, CHW), jnp.float32),
    )(xp, W1, degp)


def _zstats_body(acc_ref, degp_ref, b_ref, z_ref, st_ref, ssum, ssq, kref):
    i = pl.program_id(0)
    dinv = _dinv_of(degp_ref[...])
    z = (jnp.concatenate([acc_ref[0], acc_ref[1]], axis=1) * dinv[:, None]
         + b_ref[...])
    z_ref[...] = z

    @pl.when(i == 0)
    def _():
        kref[...] = jnp.sum(z, axis=0, keepdims=True) / float(RB)

    rid = lax.broadcasted_iota(jnp.int32, (RB, 1), 0) + i * RB
    d = jnp.where(rid < N, z - kref[...], 0.0)
    ps = jnp.sum(d, axis=0, keepdims=True)
    pq = jnp.sum(d * d, axis=0, keepdims=True)

    @pl.when(i == 0)
    def _():
        ssum[...] = ps
        ssq[...] = pq

    @pl.when(i > 0)
    def _():
        ssum[...] += ps
        ssq[...] += pq

    @pl.when(i == NBLK - 1)
    def _():
        st_ref[...] = jnp.concatenate([ssum[...], ssq[...], kref[...]],
                                      axis=0)


def _zstats(acc, degp, b):
    return pl.pallas_call(
        _zstats_body,
        grid=(NBLK,),
        in_specs=[
            pl.BlockSpec((2, RB, CHW), lambda i: (0, i, 0)),
            pl.BlockSpec((2, RB, CHW), lambda i: (0, i, 0)),
            pl.BlockSpec((1, H1), lambda i: (0, 0)),
        ],
        out_specs=[
            pl.BlockSpec((RB, H1), lambda i: (i, 0)),
            pl.BlockSpec((3, H1), lambda i: (0, 0)),
        ],
        out_shape=[
            jax.ShapeDtypeStruct((NP, H1), jnp.float32),
            jax.ShapeDtypeStruct((3, H1), jnp.float32),
        ],
        scratch_shapes=[
            pltpu.VMEM((1, H1), jnp.float32),
            pltpu.VMEM((1, H1), jnp.float32),
            pltpu.VMEM((1, H1), jnp.float32),
        ],
    )(acc, degp, b)


def _bn_relu(z, st, g, bt):
    m = st[0:1] / float(N)
    mu = m + st[2:3]
    var = st[1:2] / float(N) - m * m
    inv = lax.rsqrt(var + FEPS)
    return jnp.maximum((z - mu) * inv * g + bt, 0.0)


def _bnmm_body(z_ref, st_ref, g_ref, bt_ref, w_ref, degp_ref, out_ref):
    h = _bn_relu(z_ref[...], st_ref[...], g_ref[...], bt_ref[...])
    dinv = _dinv_of(degp_ref[...])
    y = jnp.dot(h, w_ref[...], preferred_element_type=jnp.float32)
    y = y * dinv[:, None]
    out_ref[0] = y[:, :CHW]
    out_ref[1] = y[:, CHW:]


def _bnmm(z, st, g, bt, W2, degp):
    return pl.pallas_call(
        _bnmm_body,
        grid=(NBLK,),
        in_specs=[
            pl.BlockSpec((RB, H1), lambda i: (i, 0)),
            pl.BlockSpec((3, H1), lambda i: (0, 0)),
            pl.BlockSpec((1, H1), lambda i: (0, 0)),
            pl.BlockSpec((1, H1), lambda i: (0, 0)),
            pl.BlockSpec((H1, H1), lambda i: (0, 0)),
            pl.BlockSpec((2, RB, CHW), lambda i: (0, i, 0)),
        ],
        out_specs=pl.BlockSpec((2, RB, CHW), lambda i: (0, i, 0)),
        out_shape=jax.ShapeDtypeStruct((2, NP, CHW), jnp.float32),
    )(z, st, g, bt, W2, degp)


def _pool_body(z_ref, st_ref, g_ref, bt_ref, batch_ref, ps_ref, pc_ref,
               asum, acnt):
    i = pl.program_id(0)
    h = _bn_relu(z_ref[...], st_ref[...], g_ref[...], bt_ref[...])
    b = batch_ref[0, 0, :]
    oh = (lax.broadcasted_iota(jnp.int32, (G, RB), 0) == b[None, :]).astype(
        jnp.float32)
    ps = jnp.dot(oh, h, preferred_element_type=jnp.float32,
                 precision=lax.Precision.HIGHEST)
    pc = jnp.dot(oh, jnp.ones((RB, CHW), jnp.float32),
                 preferred_element_type=jnp.float32)

    @pl.when(i == 0)
    def _():
        asum[...] = ps
        acnt[...] = pc

    @pl.when(i > 0)
    def _():
        asum[...] += ps
        acnt[...] += pc

    @pl.when(i == NBLK - 1)
    def _():
        ps_ref[...] = asum[...]
        pc_ref[...] = acnt[...]


def _pool(z, st, g, bt, batch3):
    return pl.pallas_call(
        _pool_body,
        grid=(NBLK,),
        in_specs=[
            pl.BlockSpec((RB, H1), lambda i: (i, 0)),
            pl.BlockSpec((3, H1), lambda i: (0, 0)),
            pl.BlockSpec((1, H1), lambda i: (0, 0)),
            pl.BlockSpec((1, H1), lambda i: (0, 0)),
            pl.BlockSpec((1, 1, RB), lambda i: (i, 0, 0)),
        ],
        out_specs=[
            pl.BlockSpec((G, H1), lambda i: (0, 0)),
            pl.BlockSpec((G, CHW), lambda i: (0, 0)),
        ],
        out_shape=[
            jax.ShapeDtypeStruct((G, H1), jnp.float32),
            jax.ShapeDtypeStruct((G, CHW), jnp.float32),
        ],
        scratch_shapes=[
            pltpu.VMEM((G, H1), jnp.float32),
            pltpu.VMEM((G, CHW), jnp.float32),
        ],
    )(z, st, g, bt, batch3)


def _head_body(ps_ref, pc_ref, wf2_ref, bf2_ref, g3_ref, b3_ref, wf3_ref,
               bf3_ref, out_ref, hg_ref):
    cnt = pc_ref[...][:, 0:1]
    hg = ps_ref[...] / jnp.maximum(cnt, 1.0)
    hg_ref[...] = hg
    a = jnp.dot(hg, wf2_ref[...], preferred_element_type=jnp.float32)
    a = a + bf2_ref[...]
    mu = jnp.mean(a, axis=0, keepdims=True)
    ac = a - mu
    var = jnp.mean(ac * ac, axis=0, keepdims=True)
    h2 = jnp.maximum((a - mu) * lax.rsqrt(var + FEPS) * g3_ref[...]
                     + b3_ref[...], 0.0)
    out_ref[...] = (jnp.dot(h2, wf3_ref[...], preferred_element_type=jnp.float32)
                    + bf3_ref[...])


def _head(ps, pc, Wf2, bf2, g3, b3, Wf3, bf3):
    return pl.pallas_call(
        _head_body,
        out_shape=[
            jax.ShapeDtypeStruct((G, DIM_OUT), jnp.float32),
            jax.ShapeDtypeStruct((G, H1), jnp.float32),
        ],
    )(ps, pc, Wf2, bf2, g3, b3, Wf3, bf3)


def kernel(x, edge_index, batch, W1, b1, gamma1, beta1, W2, b2, gamma2, beta2,
           Wf2, bf2, gamma3, beta3, Wf3, bf3):
    src = edge_index[0]
    dst = edge_index[1]
    srcp = jnp.concatenate([src, jnp.zeros((PAD_E,), jnp.int32)])
    dstp = jnp.concatenate([dst, jnp.full((PAD_E,), N, jnp.int32)])
    src_r = srcp.reshape(16, NCH_T, CHW)
    src2 = jnp.concatenate([src_r, src_r + NP], axis=0).reshape(
        32 * NCH_T, CHW)
    dst2 = dstp.reshape(16 * NCH_T, CHW)
    xp = jnp.pad(x, ((0, NP - N), (0, 0)))
    batch3 = jnp.pad(batch, (0, NP - N), constant_values=G).reshape(
        NBLK, 1, RB)
    zeros_h = jnp.zeros((RB, CHW), jnp.float32)
    ones_h = jnp.ones((CHW, CHW), jnp.float32)

    hist_k, mp_k = _sc_kernels()
    degp = hist_k(dst2, zeros_h, ones_h).reshape(2, NP, CHW)
    y1 = _mm1(xp, W1, degp)
    acc1 = mp_k(y1.reshape(2 * NP, CHW), src2, dst2).reshape(2, NP, CHW)
    z1, st1 = _zstats(acc1, degp, b1.reshape(1, H1))
    y2 = _bnmm(z1, st1, gamma1.reshape(1, H1), beta1.reshape(1, H1), W2, degp)
    acc2 = mp_k(y2.reshape(2 * NP, CHW), src2, dst2).reshape(2, NP, CHW)
    z2, st2 = _zstats(acc2, degp, b2.reshape(1, H1))
    ps, pc = _pool(z2, st2, gamma2.reshape(1, H1), beta2.reshape(1, H1),
                   batch3)
    out, hg = _head(ps, pc, Wf2, bf2.reshape(1, H2), gamma3.reshape(1, H2),
                    beta3.reshape(1, H2), Wf3, bf3.reshape(1, DIM_OUT))
    return (out, hg)
